# baseline (speedup 1.0000x reference)
import functools

import jax
import jax.numpy as jnp
from jax import lax
from jax.experimental import pallas as pl
from jax.experimental.pallas import tpu as pltpu
from jax.experimental.pallas import tpu_sc as plsc

B_DIM, T_DIM = 4096, 200
B = B_DIM * T_DIM
D = 64
NC, NS = 2, 16
NW = NC * NS
B_W = B // NW
CHUNK = 128
G = 4
ROWS_G = CHUNK * G
N_GRP = B_W // ROWS_G
C_W = B_W // CHUNK


def _sc_gather(idx2d, table):
    mesh = plsc.VectorSubcoreMesh(core_axis_name="c", subcore_axis_name="s")

    @functools.partial(
        pl.kernel,
        out_type=jax.ShapeDtypeStruct((B, D), jnp.float32),
        mesh=mesh,
        compiler_params=pltpu.CompilerParams(use_tc_tiling_on_sc=False),
        scratch_types=[
            pltpu.VMEM((G, CHUNK), jnp.int32),
            pltpu.VMEM((ROWS_G, D), jnp.float32),
            pltpu.SemaphoreType.DMA,
        ],
    )
    def k(idx_hbm, table_hbm, out_hbm, idx_v, rows_v, sem):
        wid = lax.axis_index("s") * NC + lax.axis_index("c")
        row_base = wid * B_W
        chunk_base = wid * C_W

        @pl.loop(0, N_GRP)
        def _(g):
            pltpu.sync_copy(idx_hbm.at[pl.ds(chunk_base + g * G, G)], idx_v)
            copies = [
                pltpu.async_copy(
                    table_hbm.at[idx_v.at[j]],
                    rows_v.at[pl.ds(j * CHUNK, CHUNK)],
                    sem,
                )
                for j in range(G)
            ]
            for c in copies:
                c.wait()
            pltpu.sync_copy(
                rows_v, out_hbm.at[pl.ds(row_base + g * ROWS_G, ROWS_G)]
            )

    return k(idx2d, table)


def kernel(x, table):
    idx = x[:, :, 1].astype(jnp.int32).reshape(B // CHUNK, CHUNK)
    out = _sc_gather(idx, table)
    return out.reshape(B_DIM, T_DIM, D)


# speedup vs baseline: 1.0384x; 1.0384x over previous
import functools

import jax
import jax.numpy as jnp
from jax import lax
from jax.experimental import pallas as pl
from jax.experimental.pallas import tpu as pltpu
from jax.experimental.pallas import tpu_sc as plsc

B_DIM, T_DIM = 4096, 200
B = B_DIM * T_DIM
D = 64
NC, NS = 2, 16
NW = NC * NS
B_W = B // NW
CHUNK = 128
C_W = B_W // CHUNK
G = 5
ROWS_G = CHUNK * G
N_GRP = C_W // G
N_PAIR = N_GRP // 2


def _sc_gather(idx2d, table):
    mesh = plsc.VectorSubcoreMesh(core_axis_name="c", subcore_axis_name="s")

    @functools.partial(
        pl.kernel,
        out_type=jax.ShapeDtypeStruct((B, D), jnp.float32),
        mesh=mesh,
        compiler_params=pltpu.CompilerParams(use_tc_tiling_on_sc=False),
        scratch_types=[
            pltpu.VMEM((C_W, CHUNK), jnp.int32),
            pltpu.VMEM((2, ROWS_G, D), jnp.float32),
            pltpu.SemaphoreType.DMA,
            pltpu.SemaphoreType.DMA,
            pltpu.SemaphoreType.DMA,
            pltpu.SemaphoreType.DMA,
        ],
    )
    def k(idx_hbm, table_hbm, out_hbm, idx_v, rows_v, gs0, gs1, os0, os1):
        wid = lax.axis_index("s") * NC + lax.axis_index("c")
        row_base = wid * B_W
        chunk_base = wid * C_W
        gsem = (gs0, gs1)
        osem = (os0, os1)

        pltpu.sync_copy(idx_hbm.at[pl.ds(chunk_base, C_W)], idx_v)

        def out_slice(t, b):
            return out_hbm.at[pl.ds(row_base + (2 * t + b) * ROWS_G, ROWS_G)]

        def fire_gathers(t, b):
            return [
                pltpu.async_copy(
                    table_hbm.at[idx_v.at[(2 * t + b) * G + j]],
                    rows_v.at[b, pl.ds(j * CHUNK, CHUNK)],
                    gsem[b],
                )
                for j in range(G)
            ]

        @pl.loop(0, N_PAIR)
        def _(t):
            @pl.when(t > 0)
            def _():
                pltpu.make_async_copy(rows_v.at[0], out_slice(t, 0), osem[0]).wait()

            d0 = fire_gathers(t, 0)

            @pl.when(t > 0)
            def _():
                pltpu.make_async_copy(rows_v.at[1], out_slice(t, 1), osem[1]).wait()

            d1 = fire_gathers(t, 1)
            for c in d0:
                c.wait()
            pltpu.async_copy(rows_v.at[0], out_slice(t, 0), osem[0])
            for c in d1:
                c.wait()
            pltpu.async_copy(rows_v.at[1], out_slice(t, 1), osem[1])

        for b in range(2):
            pltpu.make_async_copy(
                rows_v.at[b], out_slice(N_PAIR - 1, b), osem[b]
            ).wait()

    return k(idx2d, table)


def kernel(x, table):
    idx = x[:, :, 1].astype(jnp.int32).reshape(B // CHUNK, CHUNK)
    out = _sc_gather(idx, table)
    return out.reshape(B_DIM, T_DIM, D)


# speedup vs baseline: 1.4798x; 1.4251x over previous
import functools

import jax
import jax.numpy as jnp
from jax import lax
from jax.experimental import pallas as pl
from jax.experimental.pallas import tpu as pltpu
from jax.experimental.pallas import tpu_sc as plsc

VOCAB_ROWS = 1000000
B_DIM, T_DIM = 4096, 200
B = B_DIM * T_DIM
D = 64
NC, NS = 2, 16
NW = NC * NS
B_W = B // NW
CHUNK = 128
C_W = B_W // CHUNK
G = 5
ROWS_G = CHUNK * G
N_GRP = C_W // G
N_PAIR = N_GRP // 2


def _sc_gather(idx2d, table2m):
    mesh = plsc.VectorSubcoreMesh(core_axis_name="c", subcore_axis_name="s")

    @functools.partial(
        pl.kernel,
        out_type=jax.ShapeDtypeStruct((B, 2 * D), jnp.float32),
        mesh=mesh,
        compiler_params=pltpu.CompilerParams(use_tc_tiling_on_sc=False),
        scratch_types=[
            pltpu.VMEM((C_W, CHUNK), jnp.int32),
            pltpu.VMEM((2, ROWS_G, D), jnp.float32),
            pltpu.SemaphoreType.DMA,
            pltpu.SemaphoreType.DMA,
            pltpu.SemaphoreType.DMA,
            pltpu.SemaphoreType.DMA,
        ],
    )
    def k(idx_hbm, table_hbm, out_hbm, idx_v, rows_v, gs0, gs1, os0, os1):
        wid = lax.axis_index("s") * NC + lax.axis_index("c")
        row_base = wid * B_W
        chunk_base = wid * C_W
        gsem = (gs0, gs1)
        osem = (os0, os1)

        pltpu.sync_copy(idx_hbm.at[pl.ds(chunk_base, C_W)], idx_v)

        def out_slice(t, b):
            return out_hbm.at[
                pl.ds(row_base + (2 * t + b) * ROWS_G, ROWS_G), pl.ds(0, D)
            ]

        def fire_gathers(t, b):
            return [
                pltpu.async_copy(
                    table_hbm.at[idx_v.at[(2 * t + b) * G + j]],
                    rows_v.at[b, pl.ds(j * CHUNK, CHUNK)],
                    gsem[b],
                )
                for j in range(G)
            ]

        @pl.loop(0, N_PAIR)
        def _(t):
            @pl.when(t > 0)
            def _():
                pltpu.make_async_copy(rows_v.at[0], out_slice(t, 0), osem[0]).wait()

            d0 = fire_gathers(t, 0)

            @pl.when(t > 0)
            def _():
                pltpu.make_async_copy(rows_v.at[1], out_slice(t, 1), osem[1]).wait()

            d1 = fire_gathers(t, 1)
            for c in d0:
                c.wait()
            pltpu.async_copy(rows_v.at[0], out_slice(t, 0), osem[0])
            for c in d1:
                c.wait()
            pltpu.async_copy(rows_v.at[1], out_slice(t, 1), osem[1])

        for b in range(2):
            pltpu.make_async_copy(
                rows_v.at[b], out_slice(N_PAIR - 1, b), osem[b]
            ).wait()

    return k(idx2d, table2m)


def kernel(x, table):
    tpad = jnp.pad(table, ((0, 0), (0, D)))
    table2m = tpad.reshape(2 * VOCAB_ROWS, D)
    idx2 = (x[:, :, 1].astype(jnp.int32) * 2).reshape(B // CHUNK, CHUNK)
    out = _sc_gather(idx2, table2m)
    return out[:, :D].reshape(B_DIM, T_DIM, D)
